# baseline (speedup 1.0000x reference)
import functools

import jax
import jax.numpy as jnp
from jax import lax
from jax.experimental import pallas as pl
from jax.experimental.pallas import tpu as pltpu
from jax.experimental.pallas import tpu_sc as plsc

BATCH = 16384
D = 32
HID = 64
NC, NS = 2, 16
NW = NC * NS
ROWS_PER_W = BATCH // NW
CHUNK = 128
NCHUNK = ROWS_PER_W // CHUNK
IDX_ROWS = BATCH // CHUNK


def _gather_body(uidx_hbm, midx_hbm, u_table, m_table, u_out, m_out,
                 uidx_v, midx_v, urows_v, mrows_v, sem):
    wid = lax.axis_index("s") * NC + lax.axis_index("c")
    base = wid * NCHUNK
    pltpu.sync_copy(uidx_hbm.at[pl.ds(base, NCHUNK)], uidx_v)
    pltpu.sync_copy(midx_hbm.at[pl.ds(base, NCHUNK)], midx_v)
    copies = []
    for j in range(NCHUNK):
        copies.append(pltpu.async_copy(u_table.at[uidx_v.at[j]], urows_v.at[j], sem))
        copies.append(pltpu.async_copy(m_table.at[midx_v.at[j]], mrows_v.at[j], sem))
    for c in copies:
        c.wait()
    pltpu.sync_copy(urows_v, u_out.at[pl.ds(base, NCHUNK)])
    pltpu.sync_copy(mrows_v, m_out.at[pl.ds(base, NCHUNK)])


def _sc_gather(uidx, midx, u_table, m_table):
    mesh = plsc.VectorSubcoreMesh(core_axis_name="c", subcore_axis_name="s",
                                  num_cores=NC, num_subcores=NS)
    out_t = (jax.ShapeDtypeStruct((IDX_ROWS, CHUNK, D), jnp.float32),
             jax.ShapeDtypeStruct((IDX_ROWS, CHUNK, D), jnp.float32))
    scratch = [
        pltpu.VMEM((NCHUNK, CHUNK), jnp.int32),
        pltpu.VMEM((NCHUNK, CHUNK), jnp.int32),
        pltpu.VMEM((NCHUNK, CHUNK, D), jnp.float32),
        pltpu.VMEM((NCHUNK, CHUNK, D), jnp.float32),
        pltpu.SemaphoreType.DMA,
    ]
    params = pltpu.CompilerParams(use_tc_tiling_on_sc=False)
    return pl.kernel(_gather_body, out_type=out_t, mesh=mesh,
                     scratch_types=scratch,
                     compiler_params=params)(uidx, midx, u_table, m_table)


def _mlp_body(u_ref, m_ref, w1_ref, b1_ref, w2_ref, b2_ref, o_ref):
    h = jnp.dot(u_ref[...], w1_ref[0:D, :], preferred_element_type=jnp.float32)
    h = h + jnp.dot(m_ref[...], w1_ref[D:2 * D, :],
                    preferred_element_type=jnp.float32)
    h = jnp.maximum(h + b1_ref[...], 0.0)
    o = jnp.dot(h, w2_ref[...], preferred_element_type=jnp.float32) + b2_ref[...]
    o_ref[...] = jax.nn.sigmoid(o) * 5.5


def _mlp(u_emb, m_emb, w1, b1, w2, b2, block_rows=2048):
    grid = (BATCH // block_rows,)
    return pl.pallas_call(
        _mlp_body,
        grid=grid,
        in_specs=[
            pl.BlockSpec((block_rows, D), lambda i: (i, 0)),
            pl.BlockSpec((block_rows, D), lambda i: (i, 0)),
            pl.BlockSpec((2 * D, HID), lambda i: (0, 0)),
            pl.BlockSpec((1, HID), lambda i: (0, 0)),
            pl.BlockSpec((HID, 1), lambda i: (0, 0)),
            pl.BlockSpec((1, 1), lambda i: (0, 0)),
        ],
        out_specs=pl.BlockSpec((block_rows, 1), lambda i: (i, 0)),
        out_shape=jax.ShapeDtypeStruct((BATCH, 1), jnp.float32),
    )(u_emb, m_emb, w1, b1.reshape(1, HID), w2, b2.reshape(1, 1))


def kernel(cats, u_table, m_table, w1, b1, w2, b2):
    cats = cats.astype(jnp.int32)
    uidx = cats[:, 0].reshape(IDX_ROWS, CHUNK)
    midx = cats[:, 1].reshape(IDX_ROWS, CHUNK)
    u_emb, m_emb = _sc_gather(uidx, midx, u_table, m_table)
    u_emb = u_emb.reshape(BATCH, D)
    m_emb = m_emb.reshape(BATCH, D)
    return _mlp(u_emb, m_emb, w1, b1, w2, b2)
